# Initial kernel scaffold; baseline (speedup 1.0000x reference)
import functools

import jax
import jax.numpy as jnp
from jax import lax
from jax.experimental import pallas as pl
from jax.experimental.pallas import tpu as pltpu
from jax.experimental.pallas import tpu_sc as plsc

N = 10000
D = 128
E = 320000

NC = 2
NS = 16
NW = NC * NS

C = 128
EPT = 10240
EP = NW * EPT
CH = EPT // C
ROWS_PER_TILE = N // NS
LANES = 16
SUB = D // LANES


def _matmul_body(x_ref, w_ref, o_ref):
    o_ref[...] = jnp.dot(x_ref[...], w_ref[...], preferred_element_type=jnp.float32)


def _combine_body(p_ref, o_ref):
    o_ref[...] = jnp.maximum(p_ref[0] + p_ref[1], 0.0)


def _sc_spmm_body(before_hbm, col_hbm, row_hbm, adj_hbm, zeros_hbm, out_hbm,
                  col_v, row_v, adj_v, rows_v, acc, gsem, ssem):
    c = lax.axis_index("c")
    s = lax.axis_index("s")
    wid = s * NC + c
    zbase = pl.multiple_of(s * ROWS_PER_TILE, ROWS_PER_TILE)

    pltpu.sync_copy(zeros_hbm, acc.at[pl.ds(zbase, ROWS_PER_TILE)])
    plsc.subcore_barrier()

    base = pl.multiple_of(wid * EPT, C)

    def chunk_body(g, carry):
        off = pl.multiple_of(base + g * C, C)
        pltpu.sync_copy(col_hbm.at[pl.ds(off, C)], col_v)
        pltpu.sync_copy(row_hbm.at[pl.ds(off, C)], row_v)
        pltpu.sync_copy(adj_hbm.at[pl.ds(off, C)], adj_v)
        pltpu.async_copy(before_hbm.at[col_v], rows_v, gsem).wait()

        def edge_body(e, carry2):
            sval = adj_v[e]
            for j in range(SUB):
                sl = pl.ds(j * LANES, LANES)
                rows_v[e, sl] = rows_v[e, sl] * sval
            return carry2

        lax.fori_loop(0, C, edge_body, 0)
        pltpu.async_copy(rows_v, acc.at[row_v], ssem, add=True).wait()
        return carry

    lax.fori_loop(0, CH, chunk_body, 0)
    plsc.subcore_barrier()

    obase = pl.multiple_of(c * N + s * ROWS_PER_TILE, ROWS_PER_TILE)
    pltpu.sync_copy(acc.at[pl.ds(zbase, ROWS_PER_TILE)],
                    out_hbm.at[pl.ds(obase, ROWS_PER_TILE)])


def kernel(inputs, edge_index, adj_values, W):
    before = pl.pallas_call(
        _matmul_body,
        grid=(10,),
        in_specs=[
            pl.BlockSpec((N // 10, D), lambda i: (i, 0)),
            pl.BlockSpec((D, D), lambda i: (0, 0)),
        ],
        out_specs=pl.BlockSpec((N // 10, D), lambda i: (i, 0)),
        out_shape=jax.ShapeDtypeStruct((N, D), jnp.float32),
    )(inputs, W)

    pad = EP - E
    col = jnp.concatenate([edge_index[1], jnp.zeros((pad,), jnp.int32)])
    row = jnp.concatenate([edge_index[0], jnp.zeros((pad,), jnp.int32)])
    adj = jnp.concatenate([adj_values, jnp.zeros((pad,), jnp.float32)])
    zeros_tile = jnp.zeros((ROWS_PER_TILE, D), jnp.float32)

    mesh = plsc.VectorSubcoreMesh(core_axis_name="c", subcore_axis_name="s")
    sc_spmm = functools.partial(
        pl.kernel,
        out_type=jax.ShapeDtypeStruct((NC * N, D), jnp.float32),
        mesh=mesh,
        scratch_types=[
            pltpu.VMEM((C,), jnp.int32),
            pltpu.VMEM((C,), jnp.int32),
            pltpu.VMEM((C,), jnp.float32),
            pltpu.VMEM((C, D), jnp.float32),
            pltpu.VMEM_SHARED((N, D), jnp.float32),
            pltpu.SemaphoreType.DMA,
            pltpu.SemaphoreType.DMA,
        ],
    )(_sc_spmm_body)
    partials = sc_spmm(before, col, row, adj, zeros_tile)
    partials = partials.reshape(NC, N, D)

    out = pl.pallas_call(
        _combine_body,
        grid=(10,),
        in_specs=[pl.BlockSpec((NC, N // 10, D), lambda i: (0, i, 0))],
        out_specs=pl.BlockSpec((N // 10, D), lambda i: (i, 0)),
        out_shape=jax.ShapeDtypeStruct((N, D), jnp.float32),
    )(partials)
    return out


# speedup vs baseline: 2.7195x; 2.7195x over previous
import functools

import jax
import jax.numpy as jnp
from jax import lax
from jax.experimental import pallas as pl
from jax.experimental.pallas import tpu as pltpu
from jax.experimental.pallas import tpu_sc as plsc

N = 10000
D = 128
E = 320000

NC = 2
NS = 16
NW = NC * NS

C = 128
EPT = 10240
EP = NW * EPT
CH = EPT // C
NPAD = 10240
ROWS_PER_TILE = NPAD // NS
LANES = 16
SUB = D // LANES


def _matmul_body(x_ref, w_ref, o_ref):
    o_ref[...] = jnp.dot(x_ref[...], w_ref[...], preferred_element_type=jnp.float32)


def _combine_body(p_ref, o_ref):
    o_ref[...] = jnp.maximum(p_ref[0] + p_ref[1], 0.0)


def _sc_spmm_body(before_hbm, col_hbm, row_hbm, adj_hbm, zeros_hbm, out_hbm,
                  col_v, row_v, adj_v, rows_v, acc, gsem, ssem):
    c = lax.axis_index("c")
    s = lax.axis_index("s")
    wid = s * NC + c
    zbase = pl.multiple_of(s * ROWS_PER_TILE, ROWS_PER_TILE)

    pltpu.sync_copy(zeros_hbm, acc.at[pl.ds(zbase, ROWS_PER_TILE)])
    plsc.subcore_barrier()

    base = pl.multiple_of(wid * EPT, C)

    def chunk_body(g, carry):
        off = pl.multiple_of(base + g * C, C)
        pltpu.sync_copy(col_hbm.at[pl.ds(off, C)], col_v)
        pltpu.sync_copy(row_hbm.at[pl.ds(off, C)], row_v)
        pltpu.sync_copy(adj_hbm.at[pl.ds(off, C)], adj_v)
        pltpu.async_copy(before_hbm.at[col_v], rows_v, gsem).wait()

        def group_body(b, carry2):
            vec = adj_v[pl.ds(b * LANES, LANES)]
            for k in range(LANES):
                sval = vec[k]
                e = b * LANES + k
                for j in range(SUB):
                    sl = pl.ds(j * LANES, LANES)
                    rows_v[e, sl] = rows_v[e, sl] * sval
            return carry2

        lax.fori_loop(0, C // LANES, group_body, 0)
        pltpu.async_copy(rows_v, acc.at[row_v], ssem, add=True).wait()
        return carry

    lax.fori_loop(0, CH, chunk_body, 0)
    plsc.subcore_barrier()

    obase = pl.multiple_of(c * NPAD + s * ROWS_PER_TILE, ROWS_PER_TILE)
    pltpu.sync_copy(acc.at[pl.ds(zbase, ROWS_PER_TILE)],
                    out_hbm.at[pl.ds(obase, ROWS_PER_TILE)])


def kernel(inputs, edge_index, adj_values, W):
    before = pl.pallas_call(
        _matmul_body,
        grid=(10,),
        in_specs=[
            pl.BlockSpec((N // 10, D), lambda i: (i, 0)),
            pl.BlockSpec((D, D), lambda i: (0, 0)),
        ],
        out_specs=pl.BlockSpec((N // 10, D), lambda i: (i, 0)),
        out_shape=jax.ShapeDtypeStruct((N, D), jnp.float32),
    )(inputs, W)

    pad = EP - E
    col = jnp.concatenate([edge_index[1], jnp.zeros((pad,), jnp.int32)])
    row = jnp.concatenate([edge_index[0], jnp.zeros((pad,), jnp.int32)])
    adj = jnp.concatenate([adj_values, jnp.zeros((pad,), jnp.float32)])
    zeros_tile = jnp.zeros((ROWS_PER_TILE, D), jnp.float32)

    mesh = plsc.VectorSubcoreMesh(core_axis_name="c", subcore_axis_name="s")
    sc_spmm = functools.partial(
        pl.kernel,
        out_type=jax.ShapeDtypeStruct((NC * NPAD, D), jnp.float32),
        mesh=mesh,
        scratch_types=[
            pltpu.VMEM((C,), jnp.int32),
            pltpu.VMEM((C,), jnp.int32),
            pltpu.VMEM((C,), jnp.float32),
            pltpu.VMEM((C, D), jnp.float32),
            pltpu.VMEM_SHARED((NPAD, D), jnp.float32),
            pltpu.SemaphoreType.DMA,
            pltpu.SemaphoreType.DMA,
        ],
    )(_sc_spmm_body)
    partials = sc_spmm(before, col, row, adj, zeros_tile)
    partials = partials.reshape(NC, NPAD, D)

    out = pl.pallas_call(
        _combine_body,
        grid=(10,),
        in_specs=[pl.BlockSpec((NC, N // 10, D), lambda i: (0, i, 0))],
        out_specs=pl.BlockSpec((N // 10, D), lambda i: (i, 0)),
        out_shape=jax.ShapeDtypeStruct((N, D), jnp.float32),
    )(partials)
    return out


# speedup vs baseline: 3.7776x; 1.3891x over previous
import functools

import jax
import jax.numpy as jnp
from jax import lax
from jax.experimental import pallas as pl
from jax.experimental.pallas import tpu as pltpu
from jax.experimental.pallas import tpu_sc as plsc

N = 10000
D = 128
E = 320000

NC = 2
NS = 16
NW = NC * NS

C = 128
EPT = 10240
EP = NW * EPT
CH = EPT // C
NPAD = 10240
ROWS_PER_TILE = NPAD // NS
LANES = 16
SUB = D // LANES


def _matmul_body(x_ref, w_ref, o_ref):
    o_ref[...] = jnp.dot(x_ref[...], w_ref[...], preferred_element_type=jnp.float32)


def _combine_body(p_ref, o_ref):
    o_ref[...] = jnp.maximum(p_ref[0] + p_ref[1], 0.0)


def _sc_spmm_body(before_hbm, col_hbm, row_hbm, adj_hbm, zeros_hbm, out_hbm,
                  col_v, row_v, adj_v, rows_v, acc, gsems, ssems):
    c = lax.axis_index("c")
    s = lax.axis_index("s")
    wid = s * NC + c
    zbase = pl.multiple_of(s * ROWS_PER_TILE, ROWS_PER_TILE)

    pltpu.sync_copy(zeros_hbm, acc.at[pl.ds(zbase, ROWS_PER_TILE)])

    base = pl.multiple_of(wid * EPT, C)

    def load_chunk(g, slot):
        off = pl.multiple_of(base + g * C, C)
        pltpu.sync_copy(col_hbm.at[pl.ds(off, C)], col_v.at[slot])
        pltpu.sync_copy(row_hbm.at[pl.ds(off, C)], row_v.at[slot])
        pltpu.sync_copy(adj_hbm.at[pl.ds(off, C)], adj_v.at[slot])
        pltpu.async_copy(before_hbm.at[col_v.at[slot]], rows_v.at[slot],
                         gsems.at[slot])

    def scale_chunk(slot):
        def group_body(b, carry2):
            vec = adj_v[slot, pl.ds(b * LANES, LANES)]
            for k in range(LANES):
                sval = vec[k]
                e = b * LANES + k
                for j in range(SUB):
                    sl = pl.ds(j * LANES, LANES)
                    rows_v[slot, e, sl] = rows_v[slot, e, sl] * sval
            return carry2

        lax.fori_loop(0, C // LANES, group_body, 0)

    def wait_gather(slot):
        pltpu.make_async_copy(before_hbm.at[col_v.at[slot]], rows_v.at[slot],
                              gsems.at[slot]).wait()

    def scatter_chunk(slot):
        pltpu.async_copy(rows_v.at[slot], acc.at[row_v.at[slot]],
                         ssems.at[slot], add=True).wait()

    plsc.subcore_barrier()
    load_chunk(0, 0)

    def chunk_pair(gg, carry):
        g = gg * 2
        for b in range(2):
            nxt = 1 - b

            @pl.when(g + b + 1 < CH)
            def _():
                load_chunk(g + b + 1, nxt)
            wait_gather(b)
            scale_chunk(b)
            scatter_chunk(b)
        return carry

    lax.fori_loop(0, CH // 2, chunk_pair, 0)
    plsc.subcore_barrier()

    obase = pl.multiple_of(c * NPAD + s * ROWS_PER_TILE, ROWS_PER_TILE)
    pltpu.sync_copy(acc.at[pl.ds(zbase, ROWS_PER_TILE)],
                    out_hbm.at[pl.ds(obase, ROWS_PER_TILE)])


def kernel(inputs, edge_index, adj_values, W):
    before = pl.pallas_call(
        _matmul_body,
        grid=(10,),
        in_specs=[
            pl.BlockSpec((N // 10, D), lambda i: (i, 0)),
            pl.BlockSpec((D, D), lambda i: (0, 0)),
        ],
        out_specs=pl.BlockSpec((N // 10, D), lambda i: (i, 0)),
        out_shape=jax.ShapeDtypeStruct((N, D), jnp.float32),
    )(inputs, W)

    pad = EP - E
    col = jnp.concatenate([edge_index[1], jnp.zeros((pad,), jnp.int32)])
    row = jnp.concatenate([edge_index[0], jnp.zeros((pad,), jnp.int32)])
    adj = jnp.concatenate([adj_values, jnp.zeros((pad,), jnp.float32)])
    zeros_tile = jnp.zeros((ROWS_PER_TILE, D), jnp.float32)

    mesh = plsc.VectorSubcoreMesh(core_axis_name="c", subcore_axis_name="s")
    sc_spmm = functools.partial(
        pl.kernel,
        out_type=jax.ShapeDtypeStruct((NC * NPAD, D), jnp.float32),
        mesh=mesh,
        scratch_types=[
            pltpu.VMEM((2, C), jnp.int32),
            pltpu.VMEM((2, C), jnp.int32),
            pltpu.VMEM((2, C), jnp.float32),
            pltpu.VMEM((2, C, D), jnp.float32),
            pltpu.VMEM_SHARED((NPAD, D), jnp.float32),
            pltpu.SemaphoreType.DMA((2,)),
            pltpu.SemaphoreType.DMA((2,)),
        ],
    )(_sc_spmm_body)
    partials = sc_spmm(before, col, row, adj, zeros_tile)
    partials = partials.reshape(NC, NPAD, D)

    out = pl.pallas_call(
        _combine_body,
        grid=(10,),
        in_specs=[pl.BlockSpec((NC, N // 10, D), lambda i: (0, i, 0))],
        out_specs=pl.BlockSpec((N // 10, D), lambda i: (i, 0)),
        out_shape=jax.ShapeDtypeStruct((N, D), jnp.float32),
    )(partials)
    return out


# speedup vs baseline: 3.8509x; 1.0194x over previous
import functools

import jax
import jax.numpy as jnp
from jax import lax
from jax.experimental import pallas as pl
from jax.experimental.pallas import tpu as pltpu
from jax.experimental.pallas import tpu_sc as plsc

N = 10000
D = 128
E = 320000

NC = 2
NS = 16
NW = NC * NS

C = 128
EPT = 10240
EP = NW * EPT
CH = EPT // C
NPAD = 10240
ROWS_PER_TILE = NPAD // NS
LANES = 16
SUB = D // LANES


def _matmul_body(x_ref, w_ref, o_ref):
    o_ref[...] = jnp.dot(x_ref[...], w_ref[...], preferred_element_type=jnp.float32)


def _combine_body(p_ref, o_ref):
    o_ref[...] = jnp.maximum(p_ref[0] + p_ref[1], 0.0)


def _sc_spmm_body(packed_hbm, before_hbm, zeros_hbm, out_hbm,
                  idx_v, rows_v, acc, isems, gsems, ssems):
    c = lax.axis_index("c")
    s = lax.axis_index("s")
    wid = s * NC + c
    zbase = pl.multiple_of(s * ROWS_PER_TILE, ROWS_PER_TILE)

    pltpu.sync_copy(zeros_hbm, acc.at[pl.ds(zbase, ROWS_PER_TILE)])

    def issue_idx(g, islot):
        pltpu.async_copy(packed_hbm.at[wid, g], idx_v.at[islot],
                         isems.at[islot])

    def wait_idx(g, islot):
        pltpu.make_async_copy(packed_hbm.at[wid, g], idx_v.at[islot],
                              isems.at[islot]).wait()

    def issue_gather(islot, slot):
        pltpu.async_copy(before_hbm.at[idx_v.at[islot, 0]], rows_v.at[slot],
                         gsems.at[slot])

    def wait_gather(islot, slot):
        pltpu.make_async_copy(before_hbm.at[idx_v.at[islot, 0]],
                              rows_v.at[slot], gsems.at[slot]).wait()

    def scale_chunk(islot, slot):
        def group_body(b, carry2):
            vec = jax.lax.bitcast_convert_type(
                idx_v[islot, 2, pl.ds(b * LANES, LANES)], jnp.float32)
            for k in range(LANES):
                sval = vec[k]
                e = b * LANES + k
                for j in range(SUB):
                    sl = pl.ds(j * LANES, LANES)
                    rows_v[slot, e, sl] = rows_v[slot, e, sl] * sval
            return carry2

        lax.fori_loop(0, C // LANES, group_body, 0)

    def scatter_chunk(islot, slot):
        pltpu.async_copy(rows_v.at[slot], acc.at[idx_v.at[islot, 1]],
                         ssems.at[slot], add=True).wait()

    issue_idx(0, 0)
    issue_idx(1, 1)
    plsc.subcore_barrier()
    wait_idx(0, 0)
    issue_gather(0, 0)

    def chunk_quad(g4, carry):
        g = g4 * 4
        for u in range(4):
            gc = g + u
            islot = u
            islot1 = (u + 1) % 4
            islot2 = (u + 2) % 4
            slot = u % 2
            slot1 = (u + 1) % 2

            @pl.when(gc + 2 < CH)
            def _():
                issue_idx(gc + 2, islot2)

            @pl.when(gc + 1 < CH)
            def _():
                wait_idx(gc + 1, islot1)
                issue_gather(islot1, slot1)
            wait_gather(islot, slot)
            scale_chunk(islot, slot)
            scatter_chunk(islot, slot)
        return carry

    lax.fori_loop(0, CH // 4, chunk_quad, 0)
    plsc.subcore_barrier()

    obase = pl.multiple_of(c * NPAD + s * ROWS_PER_TILE, ROWS_PER_TILE)
    pltpu.sync_copy(acc.at[pl.ds(zbase, ROWS_PER_TILE)],
                    out_hbm.at[pl.ds(obase, ROWS_PER_TILE)])


def kernel(inputs, edge_index, adj_values, W):
    before = pl.pallas_call(
        _matmul_body,
        grid=(10,),
        in_specs=[
            pl.BlockSpec((N // 10, D), lambda i: (i, 0)),
            pl.BlockSpec((D, D), lambda i: (0, 0)),
        ],
        out_specs=pl.BlockSpec((N // 10, D), lambda i: (i, 0)),
        out_shape=jax.ShapeDtypeStruct((N, D), jnp.float32),
    )(inputs, W)

    pad = EP - E
    col = jnp.concatenate([edge_index[1], jnp.zeros((pad,), jnp.int32)])
    row = jnp.concatenate([edge_index[0], jnp.zeros((pad,), jnp.int32)])
    adj = jnp.concatenate([adj_values, jnp.zeros((pad,), jnp.float32)])
    adj_bits = jax.lax.bitcast_convert_type(adj, jnp.int32)
    packed = jnp.stack(
        [col.reshape(NW, CH, C), row.reshape(NW, CH, C),
         adj_bits.reshape(NW, CH, C)], axis=2)
    zeros_tile = jnp.zeros((ROWS_PER_TILE, D), jnp.float32)

    mesh = plsc.VectorSubcoreMesh(core_axis_name="c", subcore_axis_name="s")
    sc_spmm = functools.partial(
        pl.kernel,
        out_type=jax.ShapeDtypeStruct((NC * NPAD, D), jnp.float32),
        mesh=mesh,
        scratch_types=[
            pltpu.VMEM((4, 3, C), jnp.int32),
            pltpu.VMEM((2, C, D), jnp.float32),
            pltpu.VMEM_SHARED((NPAD, D), jnp.float32),
            pltpu.SemaphoreType.DMA((4,)),
            pltpu.SemaphoreType.DMA((2,)),
            pltpu.SemaphoreType.DMA((2,)),
        ],
    )(_sc_spmm_body)
    partials = sc_spmm(packed, before, zeros_tile)
    partials = partials.reshape(NC, NPAD, D)

    out = pl.pallas_call(
        _combine_body,
        grid=(10,),
        in_specs=[pl.BlockSpec((NC, N // 10, D), lambda i: (0, i, 0))],
        out_specs=pl.BlockSpec((N // 10, D), lambda i: (i, 0)),
        out_shape=jax.ShapeDtypeStruct((N, D), jnp.float32),
    )(partials)
    return out
